# Initial kernel scaffold; baseline (speedup 1.0000x reference)
import jax
import jax.numpy as jnp
from jax.experimental import pallas as pl


def kernel(x, edge_index, Wk, bk, Wq, bq, Wv, bv, Wagg, bagg, Wff, bff):
    raise NotImplementedError("write your pallas kernel here")


# speedup vs baseline: 3.9028x; 3.9028x over previous
import functools

import jax
import jax.numpy as jnp
from jax.experimental import pallas as pl
from jax.experimental.pallas import tpu as pltpu

H = 8
NB = 1000
EB = 1000


def _proj_body(x_ref, w_ref, b_ref, o_ref):
    o_ref[...] = (
        jnp.dot(x_ref[...], w_ref[...], preferred_element_type=jnp.float32)
        + b_ref[...]
    )


def _att_body(recv_ref, send_ref, q_ref, k_ref, att_ref, bmax_ref, qg_ref, kg_ref):
    dk = q_ref.shape[1] // H

    def gather(i, _):
        r = recv_ref[0, 0, i]
        s = send_ref[0, 0, i]
        qg_ref[pl.ds(i, 1), :] = q_ref[pl.ds(r, 1), :]
        kg_ref[pl.ds(i, 1), :] = k_ref[pl.ds(s, 1), :]
        return 0

    jax.lax.fori_loop(0, recv_ref.shape[2], gather, 0, unroll=4)
    p = qg_ref[...] * kg_ref[...]
    d_iota = jax.lax.broadcasted_iota(jnp.int32, (H * dk, H), 0)
    h_iota = jax.lax.broadcasted_iota(jnp.int32, (H * dk, H), 1)
    hsum = (d_iota // dk == h_iota).astype(jnp.float32)
    att = jnp.dot(p, hsum, preferred_element_type=jnp.float32)
    att_ref[...] = att
    bmax_ref[...] = jnp.max(att, axis=0, keepdims=True)[None]


def _sums_body(recv_ref, att_ref, bmax_ref, attn_ref, sums_ref):
    eb = pl.program_id(0)

    @pl.when(eb == 0)
    def _():
        sums_ref[...] = jnp.zeros_like(sums_ref)

    scale = 3.0 / jnp.max(bmax_ref[...])
    attn_ref[...] = jnp.exp(att_ref[...] * scale)

    def scatter(i, _):
        r = recv_ref[0, 0, i]
        sums_ref[pl.ds(r, 1), :] += attn_ref[pl.ds(i, 1), :]
        return 0

    jax.lax.fori_loop(0, recv_ref.shape[2], scatter, 0, unroll=4)


def _msg_body(
    recv_ref, send_ref, attn_ref, sums_ref, v_ref, out_ref, w_ref, wexp_ref, *,
    inv_sqrt_dk,
):
    eb = pl.program_id(0)
    hpg = attn_ref.shape[1]
    dv = v_ref.shape[1] // hpg

    @pl.when(eb == 0)
    def _():
        out_ref[...] = jnp.zeros_like(out_ref)

    def weights(i, _):
        r = recv_ref[0, 0, i]
        w_ref[pl.ds(i, 1), :] = (
            attn_ref[pl.ds(i, 1), :] / sums_ref[pl.ds(r, 1), :] * inv_sqrt_dk
        )
        return 0

    n_e = recv_ref.shape[2]
    jax.lax.fori_loop(0, n_e, weights, 0, unroll=4)

    h_iota = jax.lax.broadcasted_iota(jnp.int32, (hpg, v_ref.shape[1]), 0)
    c_iota = jax.lax.broadcasted_iota(jnp.int32, (hpg, v_ref.shape[1]), 1)
    sel = (h_iota == c_iota // dv).astype(jnp.float32)
    wexp_ref[...] = jnp.dot(w_ref[...], sel, preferred_element_type=jnp.float32)

    def scatter(i, _):
        r = recv_ref[0, 0, i]
        s = send_ref[0, 0, i]
        out_ref[pl.ds(r, 1), :] += wexp_ref[pl.ds(i, 1), :] * v_ref[pl.ds(s, 1), :]
        return 0

    jax.lax.fori_loop(0, n_e, scatter, 0, unroll=4)


def _dense_body(out_ref, waggt_ref, bagg_ref, wfft_ref, bff_ref, x_ref, y_ref):
    a = jax.nn.relu(out_ref[...])
    b = jax.nn.relu(
        jnp.dot(a, waggt_ref[...], preferred_element_type=jnp.float32) + bagg_ref[...]
    )
    c = jax.nn.relu(
        jnp.dot(b, wfft_ref[...], preferred_element_type=jnp.float32) + bff_ref[...]
    )
    y_ref[...] = x_ref[...] + c


def kernel(x, edge_index, Wk, bk, Wq, bq, Wv, bv, Wagg, bagg, Wff, bff):
    n, d = x.shape
    e = edge_index.shape[1]
    hdk = Wk.shape[0]
    hdv = Wv.shape[0]
    n_nb = n // NB
    n_eb = e // EB

    wqkv_t = jnp.concatenate([Wq, Wk, Wv], axis=0).T
    bqkv = jnp.concatenate([bq, bk, bv])[None, :]

    qkv = pl.pallas_call(
        _proj_body,
        grid=(n_nb,),
        in_specs=[
            pl.BlockSpec((NB, d), lambda i: (i, 0)),
            pl.BlockSpec((d, 2 * hdk + hdv), lambda i: (0, 0)),
            pl.BlockSpec((1, 2 * hdk + hdv), lambda i: (0, 0)),
        ],
        out_specs=pl.BlockSpec((NB, 2 * hdk + hdv), lambda i: (i, 0)),
        out_shape=jax.ShapeDtypeStruct((n, 2 * hdk + hdv), jnp.float32),
    )(x, wqkv_t, bqkv)
    q = qkv[:, :hdk]
    k = qkv[:, hdk : 2 * hdk]
    v = qkv[:, 2 * hdk :]

    recv3 = edge_index[0].reshape(n_eb, 1, EB)
    send3 = edge_index[1].reshape(n_eb, 1, EB)

    idx_spec = pl.BlockSpec((1, 1, EB), lambda i: (i, 0, 0), memory_space=pltpu.SMEM)
    att, bmax = pl.pallas_call(
        _att_body,
        grid=(n_eb,),
        in_specs=[
            idx_spec,
            idx_spec,
            pl.BlockSpec((n, hdk), lambda i: (0, 0)),
            pl.BlockSpec((n, hdk), lambda i: (0, 0)),
        ],
        out_specs=[
            pl.BlockSpec((EB, H), lambda i: (i, 0)),
            pl.BlockSpec((1, 1, H), lambda i: (i, 0, 0)),
        ],
        out_shape=[
            jax.ShapeDtypeStruct((e, H), jnp.float32),
            jax.ShapeDtypeStruct((n_eb, 1, H), jnp.float32),
        ],
        scratch_shapes=[
            pltpu.VMEM((EB, hdk), jnp.float32),
            pltpu.VMEM((EB, hdk), jnp.float32),
        ],
    )(recv3, send3, q, k)

    attn, sums = pl.pallas_call(
        _sums_body,
        grid=(n_eb,),
        in_specs=[
            idx_spec,
            pl.BlockSpec((EB, H), lambda i: (i, 0)),
            pl.BlockSpec((n_eb, 1, H), lambda i: (0, 0, 0)),
        ],
        out_specs=[
            pl.BlockSpec((EB, H), lambda i: (i, 0)),
            pl.BlockSpec((n, H), lambda i: (0, 0)),
        ],
        out_shape=[
            jax.ShapeDtypeStruct((e, H), jnp.float32),
            jax.ShapeDtypeStruct((n, H), jnp.float32),
        ],
    )(recv3, att, bmax)

    hgs = 2
    hpg = H // hgs
    hdvg = hdv // hgs
    out_halves = []
    for g in range(hgs):
        out_halves.append(
            pl.pallas_call(
                functools.partial(_msg_body, inv_sqrt_dk=1.0 / float(hdk // H) ** 0.5),
                grid=(n_eb,),
                in_specs=[
                    idx_spec,
                    idx_spec,
                    pl.BlockSpec((EB, hpg), lambda i: (i, 0)),
                    pl.BlockSpec((n, hpg), lambda i: (0, 0)),
                    pl.BlockSpec((n, hdvg), lambda i: (0, 0)),
                ],
                out_specs=pl.BlockSpec((n, hdvg), lambda i: (0, 0)),
                out_shape=jax.ShapeDtypeStruct((n, hdvg), jnp.float32),
                scratch_shapes=[
                    pltpu.VMEM((EB, hpg), jnp.float32),
                    pltpu.VMEM((EB, hdvg), jnp.float32),
                ],
            )(
                recv3,
                send3,
                attn[:, g * hpg : (g + 1) * hpg],
                sums[:, g * hpg : (g + 1) * hpg],
                v[:, g * hdvg : (g + 1) * hdvg],
            )
        )
    outacc = jnp.concatenate(out_halves, axis=1)

    y = pl.pallas_call(
        _dense_body,
        grid=(n_nb,),
        in_specs=[
            pl.BlockSpec((NB, hdv), lambda i: (i, 0)),
            pl.BlockSpec((hdv, d), lambda i: (0, 0)),
            pl.BlockSpec((1, d), lambda i: (0, 0)),
            pl.BlockSpec((d, d), lambda i: (0, 0)),
            pl.BlockSpec((1, d), lambda i: (0, 0)),
            pl.BlockSpec((NB, d), lambda i: (i, 0)),
        ],
        out_specs=pl.BlockSpec((NB, d), lambda i: (i, 0)),
        out_shape=jax.ShapeDtypeStruct((n, d), jnp.float32),
    )(outacc, Wagg.T, bagg[None, :], Wff.T, bff[None, :], x)
    return y


# speedup vs baseline: 4.2649x; 1.0928x over previous
import functools

import jax
import jax.numpy as jnp
from jax import lax
from jax.experimental import pallas as pl
from jax.experimental.pallas import tpu as pltpu
from jax.experimental.pallas import tpu_sc as plsc

H = 8
NB = 1000
EB = 1000
NC = 2
NS = 16
SC_C = 256


def _proj_body(x_ref, w_ref, b_ref, o_ref):
    o_ref[...] = (
        jnp.dot(x_ref[...], w_ref[...], preferred_element_type=jnp.float32)
        + b_ref[...]
    )


def _att_body(recv_ref, send_ref, q_ref, k_ref, att_ref, bmax_ref, qg_ref, kg_ref):
    dk = q_ref.shape[1] // H

    def gather(i, _):
        r = recv_ref[0, 0, i]
        s = send_ref[0, 0, i]
        qg_ref[pl.ds(i, 1), :] = q_ref[pl.ds(r, 1), :]
        kg_ref[pl.ds(i, 1), :] = k_ref[pl.ds(s, 1), :]
        return 0

    jax.lax.fori_loop(0, recv_ref.shape[2], gather, 0, unroll=4)
    p = qg_ref[...] * kg_ref[...]
    d_iota = jax.lax.broadcasted_iota(jnp.int32, (H * dk, H), 0)
    h_iota = jax.lax.broadcasted_iota(jnp.int32, (H * dk, H), 1)
    hsum = (d_iota // dk == h_iota).astype(jnp.float32)
    att = jnp.dot(p, hsum, preferred_element_type=jnp.float32)
    att_ref[...] = att
    bmax_ref[...] = jnp.max(att, axis=0, keepdims=True)[None]


def _sums_body(recv_ref, att_ref, bmax_ref, attn_ref, sums_ref, rsums_ref, *,
               inv_sqrt_dk):
    eb = pl.program_id(0)

    @pl.when(eb == 0)
    def _():
        sums_ref[...] = jnp.zeros_like(sums_ref)

    scale = 3.0 / jnp.max(bmax_ref[...])
    attn_ref[...] = jnp.exp(att_ref[...] * scale)

    def scatter(i, _):
        r = recv_ref[0, 0, i]
        sums_ref[pl.ds(r, 1), :] += attn_ref[pl.ds(i, 1), :]
        return 0

    jax.lax.fori_loop(0, recv_ref.shape[2], scatter, 0, unroll=4)

    @pl.when(eb == pl.num_programs(0) - 1)
    def _():
        rsums_ref[...] = inv_sqrt_dk / sums_ref[...]


def _sc_msg_body(vt_ref, recv_ref, send_ref, attn_ref, rs_hbm, out_ref,
                 rs_v, recv_v, send_v, idx_v, aw_v, wbuf, rows_v, outsh, sem, *,
                 n, dv, per_w):
    c = lax.axis_index("c")
    s = lax.axis_index("s")
    wbase = (c * NS + s) * per_w
    nr0 = (n // NS) & ~7
    nlast = n - nr0 * (NS - 1)
    assert SC_C % 8 == 0 and nr0 % 8 == 0 and nlast % 8 == 0
    base_row = s * nr0
    n_chunks = per_w // SC_C
    zeros16 = jnp.zeros((16,), jnp.float32)
    lane8 = dv // 16

    def head_body(h, _):
        pltpu.sync_copy(rs_hbm.at[h], rs_v)

        def zrows(i, _):
            for kk in range(lane8):
                rows_v[i, pl.ds(kk * 16, 16)] = zeros16
            return 0

        jax.lax.fori_loop(0, SC_C, zrows, 0)

        def zero_range(total):
            off = 0
            while off < total:
                sz = min(SC_C, total - off)
                pltpu.sync_copy(
                    rows_v.at[pl.ds(0, sz)],
                    outsh.at[pl.ds(base_row + off, sz)],
                )
                off += sz

        @pl.when(s < NS - 1)
        def _():
            zero_range(nr0)

        @pl.when(s == NS - 1)
        def _():
            zero_range(nlast)

        plsc.subcore_barrier()

        def chunk_body(g, _):
            base = wbase + g * SC_C
            pltpu.sync_copy(recv_ref.at[pl.ds(base, SC_C)], recv_v)
            pltpu.sync_copy(send_ref.at[pl.ds(base, SC_C)], send_v)
            pltpu.sync_copy(attn_ref.at[h, pl.ds(base, SC_C)], aw_v)
            hn = h * n

            def mkidx(g2, _):
                sv = send_v[pl.ds(g2 * 16, 16)]
                idx_v[pl.ds(g2 * 16, 16)] = sv + hn
                return 0

            jax.lax.fori_loop(0, SC_C // 16, mkidx, 0)
            pltpu.async_copy(vt_ref.at[idx_v], rows_v, sem).wait()

            def scale_grp(g2, _):
                rcv16 = recv_v[pl.ds(g2 * 16, 16)]
                rs16 = plsc.load_gather(rs_v, [rcv16])
                aw16 = aw_v[pl.ds(g2 * 16, 16)]
                wbuf[...] = aw16 * rs16

                def scale_e(j, _):
                    wj = plsc.load_gather(wbuf, [jnp.broadcast_to(j, (16,))])
                    e = g2 * 16 + j
                    for kk in range(lane8):
                        rows_v[e, pl.ds(kk * 16, 16)] *= wj
                    return 0

                jax.lax.fori_loop(0, 16, scale_e, 0)
                return 0

            jax.lax.fori_loop(0, SC_C // 16, scale_grp, 0)
            pltpu.sync_copy(rows_v, outsh.at[recv_v], add=True)
            return 0

        jax.lax.fori_loop(0, n_chunks, chunk_body, 0)
        plsc.subcore_barrier()

        @pl.when(s < NS - 1)
        def _():
            pltpu.sync_copy(
                outsh.at[pl.ds(base_row, nr0)],
                out_ref.at[c, h, pl.ds(base_row, nr0)],
            )

        @pl.when(s == NS - 1)
        def _():
            pltpu.sync_copy(
                outsh.at[pl.ds(base_row, nlast)],
                out_ref.at[c, h, pl.ds(base_row, nlast)],
            )

        plsc.subcore_barrier()
        return 0

    jax.lax.fori_loop(0, H, head_body, 0)


def _msg_body(
    recv_ref, send_ref, attn_ref, sums_ref, v_ref, out_ref, w_ref, wexp_ref, *,
    inv_sqrt_dk,
):
    eb = pl.program_id(0)
    hpg = attn_ref.shape[1]
    dv = v_ref.shape[1] // hpg

    @pl.when(eb == 0)
    def _():
        out_ref[...] = jnp.zeros_like(out_ref)

    def weights(i, _):
        r = recv_ref[0, 0, i]
        w_ref[pl.ds(i, 1), :] = (
            attn_ref[pl.ds(i, 1), :] / sums_ref[pl.ds(r, 1), :] * inv_sqrt_dk
        )
        return 0

    n_e = recv_ref.shape[2]
    jax.lax.fori_loop(0, n_e, weights, 0, unroll=4)

    h_iota = jax.lax.broadcasted_iota(jnp.int32, (hpg, v_ref.shape[1]), 0)
    c_iota = jax.lax.broadcasted_iota(jnp.int32, (hpg, v_ref.shape[1]), 1)
    sel = (h_iota == c_iota // dv).astype(jnp.float32)
    wexp_ref[...] = jnp.dot(w_ref[...], sel, preferred_element_type=jnp.float32)

    def scatter(i, _):
        r = recv_ref[0, 0, i]
        s = send_ref[0, 0, i]
        out_ref[pl.ds(r, 1), :] += wexp_ref[pl.ds(i, 1), :] * v_ref[pl.ds(s, 1), :]
        return 0

    jax.lax.fori_loop(0, n_e, scatter, 0, unroll=4)


def _dense_body(p0_ref, p1_ref, waggt_ref, bagg_ref, wfft_ref, bff_ref, x_ref, y_ref):
    a = jax.nn.relu(p0_ref[...] + p1_ref[...])
    b = jax.nn.relu(
        jnp.dot(a, waggt_ref[...], preferred_element_type=jnp.float32) + bagg_ref[...]
    )
    c = jax.nn.relu(
        jnp.dot(b, wfft_ref[...], preferred_element_type=jnp.float32) + bff_ref[...]
    )
    y_ref[...] = x_ref[...] + c


def kernel(x, edge_index, Wk, bk, Wq, bq, Wv, bv, Wagg, bagg, Wff, bff):
    n, d = x.shape
    e = edge_index.shape[1]
    hdk = Wk.shape[0]
    hdv = Wv.shape[0]
    n_nb = n // NB
    n_eb = e // EB

    wqkv_t = jnp.concatenate([Wq, Wk, Wv], axis=0).T
    bqkv = jnp.concatenate([bq, bk, bv])[None, :]

    qkv = pl.pallas_call(
        _proj_body,
        grid=(n_nb,),
        in_specs=[
            pl.BlockSpec((NB, d), lambda i: (i, 0)),
            pl.BlockSpec((d, 2 * hdk + hdv), lambda i: (0, 0)),
            pl.BlockSpec((1, 2 * hdk + hdv), lambda i: (0, 0)),
        ],
        out_specs=pl.BlockSpec((NB, 2 * hdk + hdv), lambda i: (i, 0)),
        out_shape=jax.ShapeDtypeStruct((n, 2 * hdk + hdv), jnp.float32),
    )(x, wqkv_t, bqkv)
    q = qkv[:, :hdk]
    k = qkv[:, hdk : 2 * hdk]
    v = qkv[:, 2 * hdk :]

    recv3 = edge_index[0].reshape(n_eb, 1, EB)
    send3 = edge_index[1].reshape(n_eb, 1, EB)

    idx_spec = pl.BlockSpec((1, 1, EB), lambda i: (i, 0, 0), memory_space=pltpu.SMEM)
    att, bmax = pl.pallas_call(
        _att_body,
        grid=(n_eb,),
        in_specs=[
            idx_spec,
            idx_spec,
            pl.BlockSpec((n, hdk), lambda i: (0, 0)),
            pl.BlockSpec((n, hdk), lambda i: (0, 0)),
        ],
        out_specs=[
            pl.BlockSpec((EB, H), lambda i: (i, 0)),
            pl.BlockSpec((1, 1, H), lambda i: (i, 0, 0)),
        ],
        out_shape=[
            jax.ShapeDtypeStruct((e, H), jnp.float32),
            jax.ShapeDtypeStruct((n_eb, 1, H), jnp.float32),
        ],
        scratch_shapes=[
            pltpu.VMEM((EB, hdk), jnp.float32),
            pltpu.VMEM((EB, hdk), jnp.float32),
        ],
    )(recv3, send3, q, k)

    attn, sums, rsums = pl.pallas_call(
        functools.partial(_sums_body, inv_sqrt_dk=1.0 / float(hdk // H) ** 0.5),
        grid=(n_eb,),
        in_specs=[
            idx_spec,
            pl.BlockSpec((EB, H), lambda i: (i, 0)),
            pl.BlockSpec((n_eb, 1, H), lambda i: (0, 0, 0)),
        ],
        out_specs=[
            pl.BlockSpec((EB, H), lambda i: (i, 0)),
            pl.BlockSpec((n, H), lambda i: (0, 0)),
            pl.BlockSpec((n, H), lambda i: (0, 0)),
        ],
        out_shape=[
            jax.ShapeDtypeStruct((e, H), jnp.float32),
            jax.ShapeDtypeStruct((n, H), jnp.float32),
            jax.ShapeDtypeStruct((n, H), jnp.float32),
        ],
    )(recv3, att, bmax)

    dv = hdv // H
    grp = NC * NS * SC_C
    ep = ((e + grp - 1) // grp) * grp
    per_w = ep // (NC * NS)
    recv_p = jnp.pad(edge_index[0], (0, ep - e))
    send_p = jnp.pad(edge_index[1], (0, ep - e))
    attn_p = jnp.pad(attn, ((0, ep - e), (0, 0)))
    vt2 = v.reshape(n, H, dv).transpose(1, 0, 2).reshape(H * n, dv)

    sc_msg = pl.kernel(
        functools.partial(_sc_msg_body, n=n, dv=dv, per_w=per_w),
        out_type=jax.ShapeDtypeStruct((NC, H, n, dv), jnp.float32),
        mesh=plsc.VectorSubcoreMesh(core_axis_name="c", subcore_axis_name="s"),
        compiler_params=pltpu.CompilerParams(needs_layout_passes=False),
        scratch_types=[
            pltpu.VMEM((n,), jnp.float32),
            pltpu.VMEM((SC_C,), jnp.int32),
            pltpu.VMEM((SC_C,), jnp.int32),
            pltpu.VMEM((SC_C,), jnp.int32),
            pltpu.VMEM((SC_C,), jnp.float32),
            pltpu.VMEM((16,), jnp.float32),
            pltpu.VMEM((SC_C, dv), jnp.float32),
            pltpu.VMEM_SHARED((n, dv), jnp.float32),
            pltpu.SemaphoreType.DMA,
        ],
    )
    partials = sc_msg(vt2, recv_p, send_p, attn_p.T, rsums.T)
    outacc = partials.transpose(0, 2, 1, 3).reshape(NC, n, hdv)

    y = pl.pallas_call(
        _dense_body,
        grid=(n_nb,),
        in_specs=[
            pl.BlockSpec((NB, hdv), lambda i: (i, 0)),
            pl.BlockSpec((NB, hdv), lambda i: (i, 0)),
            pl.BlockSpec((hdv, d), lambda i: (0, 0)),
            pl.BlockSpec((1, d), lambda i: (0, 0)),
            pl.BlockSpec((d, d), lambda i: (0, 0)),
            pl.BlockSpec((1, d), lambda i: (0, 0)),
            pl.BlockSpec((NB, d), lambda i: (i, 0)),
        ],
        out_specs=pl.BlockSpec((NB, d), lambda i: (i, 0)),
        out_shape=jax.ShapeDtypeStruct((n, d), jnp.float32),
    )(outacc[0], outacc[1], Wagg.T, bagg[None, :], Wff.T, bff[None, :], x)
    return y


# speedup vs baseline: 5.0077x; 1.1742x over previous
import functools

import jax
import jax.numpy as jnp
from jax import lax
from jax.experimental import pallas as pl
from jax.experimental.pallas import tpu as pltpu
from jax.experimental.pallas import tpu_sc as plsc

H = 8
NB = 1000
EB = 1000
NC = 2
NS = 16
SC_C = 128


def _proj_body(x_ref, w_ref, b_ref, o_ref):
    o_ref[...] = (
        jnp.dot(x_ref[...], w_ref[...], preferred_element_type=jnp.float32)
        + b_ref[...]
    )


def _att_body(recv_ref, send_ref, q_ref, k_ref, att_ref, bmax_ref, qg_ref, kg_ref):
    dk = q_ref.shape[1] // H

    def gather(i, _):
        r = recv_ref[0, 0, i]
        s = send_ref[0, 0, i]
        qg_ref[pl.ds(i, 1), :] = q_ref[pl.ds(r, 1), :]
        kg_ref[pl.ds(i, 1), :] = k_ref[pl.ds(s, 1), :]
        return 0

    jax.lax.fori_loop(0, recv_ref.shape[2], gather, 0, unroll=4)
    p = qg_ref[...] * kg_ref[...]
    d_iota = jax.lax.broadcasted_iota(jnp.int32, (H * dk, H), 0)
    h_iota = jax.lax.broadcasted_iota(jnp.int32, (H * dk, H), 1)
    hsum = (d_iota // dk == h_iota).astype(jnp.float32)
    att = jnp.dot(p, hsum, preferred_element_type=jnp.float32)
    att_ref[...] = att
    bmax_ref[...] = jnp.max(att, axis=0, keepdims=True)[None]


def _sums_body(recv_ref, att_ref, bmax_ref, attn_ref, sums_ref, rsums_ref, *,
               inv_sqrt_dk):
    eb = pl.program_id(0)

    @pl.when(eb == 0)
    def _():
        sums_ref[...] = jnp.zeros_like(sums_ref)

    scale = 3.0 / jnp.max(bmax_ref[...])
    attn_ref[...] = jnp.exp(att_ref[...] * scale)

    def scatter(i, _):
        r = recv_ref[0, 0, i]
        sums_ref[pl.ds(r, 1), :] += attn_ref[pl.ds(i, 1), :]
        return 0

    jax.lax.fori_loop(0, recv_ref.shape[2], scatter, 0, unroll=4)

    @pl.when(eb == pl.num_programs(0) - 1)
    def _():
        rsums_ref[...] = inv_sqrt_dk / sums_ref[...]


def _sc_msg_body(vt_ref, ridx_ref, attn_ref, rs_hbm, out_ref,
                 rs_v, ridx0, ridx1, idx0, idx1, aw0, aw1, wbuf,
                 rows0, rows1, outsh, sem, *, n, dv, per_w):
    c = lax.axis_index("c")
    s = lax.axis_index("s")
    wbase = (c * NS + s) * per_w
    nr0 = (n // NS) & ~7
    nlast = n - nr0 * (NS - 1)
    assert SC_C % 16 == 0 and nr0 % 8 == 0 and nlast % 8 == 0
    base_row = s * nr0
    n_chunks = per_w // SC_C
    assert n_chunks % 2 == 0
    zeros16 = jnp.zeros((16,), jnp.float32)
    lane8 = dv // 16

    def head_body(h, _):
        pltpu.sync_copy(rs_hbm.at[h], rs_v)
        hn = h * n

        def zrows(i, _):
            for kk in range(lane8):
                rows0[i, pl.ds(kk * 16, 16)] = zeros16
            return 0

        jax.lax.fori_loop(0, SC_C, zrows, 0)

        def zero_range(total):
            off = 0
            while off < total:
                sz = min(SC_C, total - off)
                pltpu.sync_copy(
                    rows0.at[pl.ds(0, sz)],
                    outsh.at[pl.ds(base_row + off, sz)],
                )
                off += sz

        @pl.when(s < NS - 1)
        def _():
            zero_range(nr0)

        @pl.when(s == NS - 1)
        def _():
            zero_range(nlast)

        plsc.subcore_barrier()

        def load_edges(g, ridx_v, aw_v, idx_v):
            base = wbase + g * SC_C
            pltpu.sync_copy(ridx_ref.at[:, pl.ds(base, SC_C)], ridx_v)
            pltpu.sync_copy(attn_ref.at[h, pl.ds(base, SC_C)], aw_v)

            def mkidx(g2, _):
                sv = ridx_v[1, pl.ds(g2 * 16, 16)]
                idx_v[pl.ds(g2 * 16, 16)] = sv + hn
                return 0

            jax.lax.fori_loop(0, SC_C // 16, mkidx, 0)

        def process(ridx_v, aw_v, rows_v):
            def scale_grp(g2, _):
                rcv16 = ridx_v[0, pl.ds(g2 * 16, 16)]
                rs16 = plsc.load_gather(rs_v, [rcv16])
                aw16 = aw_v[pl.ds(g2 * 16, 16)]
                wbuf[...] = aw16 * rs16

                def scale_e(j, _):
                    wj = plsc.load_gather(wbuf, [jnp.broadcast_to(j, (16,))])
                    e = g2 * 16 + j
                    for kk in range(lane8):
                        rows_v[e, pl.ds(kk * 16, 16)] *= wj
                    return 0

                jax.lax.fori_loop(0, 16, scale_e, 0)
                return 0

            jax.lax.fori_loop(0, SC_C // 16, scale_grp, 0)
            pltpu.sync_copy(rows_v, outsh.at[ridx_v.at[0]], add=True)

        load_edges(0, ridx0, aw0, idx0)
        pltpu.async_copy(vt_ref.at[idx0], rows0, sem)

        def pair_body(t, _):
            pltpu.make_async_copy(vt_ref.at[idx0], rows0, sem).wait()
            load_edges(2 * t + 1, ridx1, aw1, idx1)
            pltpu.async_copy(vt_ref.at[idx1], rows1, sem)
            process(ridx0, aw0, rows0)
            pltpu.make_async_copy(vt_ref.at[idx1], rows1, sem).wait()

            @pl.when(t < n_chunks // 2 - 1)
            def _():
                load_edges(2 * t + 2, ridx0, aw0, idx0)
                pltpu.async_copy(vt_ref.at[idx0], rows0, sem)

            process(ridx1, aw1, rows1)
            return 0

        jax.lax.fori_loop(0, n_chunks // 2, pair_body, 0)
        plsc.subcore_barrier()

        @pl.when(s < NS - 1)
        def _():
            pltpu.sync_copy(
                outsh.at[pl.ds(base_row, nr0)],
                out_ref.at[c, h, pl.ds(base_row, nr0)],
            )

        @pl.when(s == NS - 1)
        def _():
            pltpu.sync_copy(
                outsh.at[pl.ds(base_row, nlast)],
                out_ref.at[c, h, pl.ds(base_row, nlast)],
            )

        plsc.subcore_barrier()
        return 0

    jax.lax.fori_loop(0, H, head_body, 0)


def _msg_body(
    recv_ref, send_ref, attn_ref, sums_ref, v_ref, out_ref, w_ref, wexp_ref, *,
    inv_sqrt_dk,
):
    eb = pl.program_id(0)
    hpg = attn_ref.shape[1]
    dv = v_ref.shape[1] // hpg

    @pl.when(eb == 0)
    def _():
        out_ref[...] = jnp.zeros_like(out_ref)

    def weights(i, _):
        r = recv_ref[0, 0, i]
        w_ref[pl.ds(i, 1), :] = (
            attn_ref[pl.ds(i, 1), :] / sums_ref[pl.ds(r, 1), :] * inv_sqrt_dk
        )
        return 0

    n_e = recv_ref.shape[2]
    jax.lax.fori_loop(0, n_e, weights, 0, unroll=4)

    h_iota = jax.lax.broadcasted_iota(jnp.int32, (hpg, v_ref.shape[1]), 0)
    c_iota = jax.lax.broadcasted_iota(jnp.int32, (hpg, v_ref.shape[1]), 1)
    sel = (h_iota == c_iota // dv).astype(jnp.float32)
    wexp_ref[...] = jnp.dot(w_ref[...], sel, preferred_element_type=jnp.float32)

    def scatter(i, _):
        r = recv_ref[0, 0, i]
        s = send_ref[0, 0, i]
        out_ref[pl.ds(r, 1), :] += wexp_ref[pl.ds(i, 1), :] * v_ref[pl.ds(s, 1), :]
        return 0

    jax.lax.fori_loop(0, n_e, scatter, 0, unroll=4)


def _dense_body(p0_ref, p1_ref, waggt_ref, bagg_ref, wfft_ref, bff_ref, x_ref, y_ref):
    a = jax.nn.relu(p0_ref[...] + p1_ref[...])
    b = jax.nn.relu(
        jnp.dot(a, waggt_ref[...], preferred_element_type=jnp.float32) + bagg_ref[...]
    )
    c = jax.nn.relu(
        jnp.dot(b, wfft_ref[...], preferred_element_type=jnp.float32) + bff_ref[...]
    )
    y_ref[...] = x_ref[...] + c


def kernel(x, edge_index, Wk, bk, Wq, bq, Wv, bv, Wagg, bagg, Wff, bff):
    n, d = x.shape
    e = edge_index.shape[1]
    hdk = Wk.shape[0]
    hdv = Wv.shape[0]
    n_nb = n // NB
    n_eb = e // EB

    wqkv_t = jnp.concatenate([Wq, Wk, Wv], axis=0).T
    bqkv = jnp.concatenate([bq, bk, bv])[None, :]

    qkv = pl.pallas_call(
        _proj_body,
        grid=(n_nb,),
        in_specs=[
            pl.BlockSpec((NB, d), lambda i: (i, 0)),
            pl.BlockSpec((d, 2 * hdk + hdv), lambda i: (0, 0)),
            pl.BlockSpec((1, 2 * hdk + hdv), lambda i: (0, 0)),
        ],
        out_specs=pl.BlockSpec((NB, 2 * hdk + hdv), lambda i: (i, 0)),
        out_shape=jax.ShapeDtypeStruct((n, 2 * hdk + hdv), jnp.float32),
    )(x, wqkv_t, bqkv)
    q = qkv[:, :hdk]
    k = qkv[:, hdk : 2 * hdk]
    v = qkv[:, 2 * hdk :]

    recv3 = edge_index[0].reshape(n_eb, 1, EB)
    send3 = edge_index[1].reshape(n_eb, 1, EB)

    idx_spec = pl.BlockSpec((1, 1, EB), lambda i: (i, 0, 0), memory_space=pltpu.SMEM)
    att, bmax = pl.pallas_call(
        _att_body,
        grid=(n_eb,),
        in_specs=[
            idx_spec,
            idx_spec,
            pl.BlockSpec((n, hdk), lambda i: (0, 0)),
            pl.BlockSpec((n, hdk), lambda i: (0, 0)),
        ],
        out_specs=[
            pl.BlockSpec((EB, H), lambda i: (i, 0)),
            pl.BlockSpec((1, 1, H), lambda i: (i, 0, 0)),
        ],
        out_shape=[
            jax.ShapeDtypeStruct((e, H), jnp.float32),
            jax.ShapeDtypeStruct((n_eb, 1, H), jnp.float32),
        ],
        scratch_shapes=[
            pltpu.VMEM((EB, hdk), jnp.float32),
            pltpu.VMEM((EB, hdk), jnp.float32),
        ],
    )(recv3, send3, q, k)

    attn, sums, rsums = pl.pallas_call(
        functools.partial(_sums_body, inv_sqrt_dk=1.0 / float(hdk // H) ** 0.5),
        grid=(n_eb,),
        in_specs=[
            idx_spec,
            pl.BlockSpec((EB, H), lambda i: (i, 0)),
            pl.BlockSpec((n_eb, 1, H), lambda i: (0, 0, 0)),
        ],
        out_specs=[
            pl.BlockSpec((EB, H), lambda i: (i, 0)),
            pl.BlockSpec((n, H), lambda i: (0, 0)),
            pl.BlockSpec((n, H), lambda i: (0, 0)),
        ],
        out_shape=[
            jax.ShapeDtypeStruct((e, H), jnp.float32),
            jax.ShapeDtypeStruct((n, H), jnp.float32),
            jax.ShapeDtypeStruct((n, H), jnp.float32),
        ],
    )(recv3, att, bmax)

    dv = hdv // H
    grp = NC * NS * SC_C
    ep = ((e + grp - 1) // grp) * grp
    per_w = ep // (NC * NS)
    redge = jnp.pad(edge_index, ((0, 0), (0, ep - e)))
    attn_p = jnp.pad(attn, ((0, ep - e), (0, 0)))
    vt2 = v.reshape(n, H, dv).transpose(1, 0, 2).reshape(H * n, dv)

    sc_msg = pl.kernel(
        functools.partial(_sc_msg_body, n=n, dv=dv, per_w=per_w),
        out_type=jax.ShapeDtypeStruct((NC, H, n, dv), jnp.float32),
        mesh=plsc.VectorSubcoreMesh(core_axis_name="c", subcore_axis_name="s"),
        compiler_params=pltpu.CompilerParams(needs_layout_passes=False),
        scratch_types=[
            pltpu.VMEM((n,), jnp.float32),
            pltpu.VMEM((2, SC_C), jnp.int32),
            pltpu.VMEM((2, SC_C), jnp.int32),
            pltpu.VMEM((SC_C,), jnp.int32),
            pltpu.VMEM((SC_C,), jnp.int32),
            pltpu.VMEM((SC_C,), jnp.float32),
            pltpu.VMEM((SC_C,), jnp.float32),
            pltpu.VMEM((16,), jnp.float32),
            pltpu.VMEM((SC_C, dv), jnp.float32),
            pltpu.VMEM((SC_C, dv), jnp.float32),
            pltpu.VMEM_SHARED((n, dv), jnp.float32),
            pltpu.SemaphoreType.DMA,
        ],
    )
    partials = sc_msg(vt2, redge, attn_p.T, rsums.T)
    outacc = partials.transpose(0, 2, 1, 3).reshape(NC, n, hdv)

    y = pl.pallas_call(
        _dense_body,
        grid=(n_nb,),
        in_specs=[
            pl.BlockSpec((NB, hdv), lambda i: (i, 0)),
            pl.BlockSpec((NB, hdv), lambda i: (i, 0)),
            pl.BlockSpec((hdv, d), lambda i: (0, 0)),
            pl.BlockSpec((1, d), lambda i: (0, 0)),
            pl.BlockSpec((d, d), lambda i: (0, 0)),
            pl.BlockSpec((1, d), lambda i: (0, 0)),
            pl.BlockSpec((NB, d), lambda i: (i, 0)),
        ],
        out_specs=pl.BlockSpec((NB, d), lambda i: (i, 0)),
        out_shape=jax.ShapeDtypeStruct((n, d), jnp.float32),
    )(outacc[0], outacc[1], Wagg.T, bagg[None, :], Wff.T, bff[None, :], x)
    return y


# speedup vs baseline: 5.3514x; 1.0686x over previous
import functools

import jax
import jax.numpy as jnp
from jax import lax
from jax.experimental import pallas as pl
from jax.experimental.pallas import tpu as pltpu
from jax.experimental.pallas import tpu_sc as plsc

H = 8
NB = 1000
EB = 1000
NC = 2
NS = 16
SC_C = 128


def _proj_body(x_ref, w_ref, b_ref, o_ref):
    o_ref[...] = (
        jnp.dot(x_ref[...], w_ref[...], preferred_element_type=jnp.float32)
        + b_ref[...]
    )


def _att_body(recv_ref, send_ref, q_ref, k_ref, att_ref, bmax_ref, qg_ref, kg_ref):
    dk = q_ref.shape[1] // H

    def gather(i, _):
        r = recv_ref[0, 0, i]
        s = send_ref[0, 0, i]
        qg_ref[pl.ds(i, 1), :] = q_ref[pl.ds(r, 1), :]
        kg_ref[pl.ds(i, 1), :] = k_ref[pl.ds(s, 1), :]
        return 0

    jax.lax.fori_loop(0, recv_ref.shape[2], gather, 0, unroll=8)
    p = qg_ref[...] * kg_ref[...]
    d_iota = jax.lax.broadcasted_iota(jnp.int32, (H * dk, H), 0)
    h_iota = jax.lax.broadcasted_iota(jnp.int32, (H * dk, H), 1)
    hsum = (d_iota // dk == h_iota).astype(jnp.float32)
    att = jnp.dot(p, hsum, preferred_element_type=jnp.float32)
    att_ref[...] = att
    bmax_ref[...] = jnp.max(att, axis=0, keepdims=True)[None]


def _sums_body(recv_ref, att_ref, bmax_ref, attn_ref, sums_ref, rsums_ref, *,
               inv_sqrt_dk):
    eb = pl.program_id(0)

    @pl.when(eb == 0)
    def _():
        sums_ref[...] = jnp.zeros_like(sums_ref)

    scale = 3.0 / jnp.max(bmax_ref[...])
    attn_ref[...] = jnp.exp(att_ref[...] * scale)

    def scatter(i, _):
        r = recv_ref[0, 0, i]
        sums_ref[pl.ds(r, 1), :] += attn_ref[pl.ds(i, 1), :]
        return 0

    jax.lax.fori_loop(0, recv_ref.shape[2], scatter, 0, unroll=8)

    @pl.when(eb == pl.num_programs(0) - 1)
    def _():
        rsums_ref[...] = inv_sqrt_dk / sums_ref[...]


def _sc_msg_body(vt_ref, ridx_ref, attn_ref, rs_hbm, out_ref,
                 rs_v, ridx0, ridx1, idx0, idx1, aw0, aw1, wbuf,
                 rows0, rows1, outsh, sem, *, n, dv, per_w):
    c = lax.axis_index("c")
    s = lax.axis_index("s")
    wbase = (c * NS + s) * per_w
    nr0 = (n // NS) & ~7
    nlast = n - nr0 * (NS - 1)
    assert SC_C % 16 == 0 and nr0 % 8 == 0 and nlast % 8 == 0
    base_row = s * nr0
    n_chunks = per_w // SC_C
    assert n_chunks % 2 == 0
    zeros16 = jnp.zeros((16,), jnp.float32)
    lane8 = dv // 16

    def head_body(h, _):
        pltpu.sync_copy(rs_hbm.at[h], rs_v)
        hn = h * n

        def zrows(i, _):
            for kk in range(lane8):
                rows0[i, pl.ds(kk * 16, 16)] = zeros16
            return 0

        jax.lax.fori_loop(0, SC_C, zrows, 0)

        def zero_range(total):
            off = 0
            while off < total:
                sz = min(SC_C, total - off)
                pltpu.sync_copy(
                    rows0.at[pl.ds(0, sz)],
                    outsh.at[pl.ds(base_row + off, sz)],
                )
                off += sz

        @pl.when(s < NS - 1)
        def _():
            zero_range(nr0)

        @pl.when(s == NS - 1)
        def _():
            zero_range(nlast)

        plsc.subcore_barrier()

        def load_edges(g, ridx_v, aw_v, idx_v):
            base = wbase + g * SC_C
            pltpu.sync_copy(ridx_ref.at[:, pl.ds(base, SC_C)], ridx_v)
            pltpu.sync_copy(attn_ref.at[h, pl.ds(base, SC_C)], aw_v)

            def mkidx(g2, _):
                sv = ridx_v[1, pl.ds(g2 * 16, 16)]
                idx_v[pl.ds(g2 * 16, 16)] = sv + hn
                return 0

            jax.lax.fori_loop(0, SC_C // 16, mkidx, 0)

        def process(ridx_v, aw_v, rows_v):
            def scale_grp(g2, _):
                rcv16 = ridx_v[0, pl.ds(g2 * 16, 16)]
                rs16 = plsc.load_gather(rs_v, [rcv16])
                aw16 = aw_v[pl.ds(g2 * 16, 16)]
                wbuf[...] = aw16 * rs16

                def scale_e(j, _):
                    wj = plsc.load_gather(wbuf, [jnp.broadcast_to(j, (16,))])
                    e = g2 * 16 + j
                    for kk in range(lane8):
                        rows_v[e, pl.ds(kk * 16, 16)] *= wj
                    return 0

                jax.lax.fori_loop(0, 16, scale_e, 0)
                return 0

            jax.lax.fori_loop(0, SC_C // 16, scale_grp, 0)
            pltpu.sync_copy(rows_v, outsh.at[ridx_v.at[0]], add=True)

        load_edges(0, ridx0, aw0, idx0)
        pltpu.async_copy(vt_ref.at[idx0], rows0, sem)

        def pair_body(t, _):
            pltpu.make_async_copy(vt_ref.at[idx0], rows0, sem).wait()
            load_edges(2 * t + 1, ridx1, aw1, idx1)
            pltpu.async_copy(vt_ref.at[idx1], rows1, sem)
            process(ridx0, aw0, rows0)
            pltpu.make_async_copy(vt_ref.at[idx1], rows1, sem).wait()

            @pl.when(t < n_chunks // 2 - 1)
            def _():
                load_edges(2 * t + 2, ridx0, aw0, idx0)
                pltpu.async_copy(vt_ref.at[idx0], rows0, sem)

            process(ridx1, aw1, rows1)
            return 0

        jax.lax.fori_loop(0, n_chunks // 2, pair_body, 0)
        plsc.subcore_barrier()

        @pl.when(s < NS - 1)
        def _():
            pltpu.sync_copy(
                outsh.at[pl.ds(base_row, nr0)],
                out_ref.at[c, h, pl.ds(base_row, nr0)],
            )

        @pl.when(s == NS - 1)
        def _():
            pltpu.sync_copy(
                outsh.at[pl.ds(base_row, nlast)],
                out_ref.at[c, h, pl.ds(base_row, nlast)],
            )

        plsc.subcore_barrier()
        return 0

    jax.lax.fori_loop(0, H, head_body, 0)


def _dense_body(p0_ref, p1_ref, waggt_ref, bagg_ref, wfft_ref, bff_ref, x_ref, y_ref):
    a = jax.nn.relu(p0_ref[...] + p1_ref[...])
    b = jax.nn.relu(
        jnp.dot(a, waggt_ref[...], preferred_element_type=jnp.float32) + bagg_ref[...]
    )
    c = jax.nn.relu(
        jnp.dot(b, wfft_ref[...], preferred_element_type=jnp.float32) + bff_ref[...]
    )
    y_ref[...] = x_ref[...] + c


def kernel(x, edge_index, Wk, bk, Wq, bq, Wv, bv, Wagg, bagg, Wff, bff):
    n, d = x.shape
    e = edge_index.shape[1]
    hdk = Wk.shape[0]
    hdv = Wv.shape[0]
    n_nb = n // NB
    n_eb = e // EB

    wqkv_t = jnp.concatenate([Wq, Wk, Wv], axis=0).T
    bqkv = jnp.concatenate([bq, bk, bv])[None, :]

    qkv = pl.pallas_call(
        _proj_body,
        grid=(n_nb,),
        in_specs=[
            pl.BlockSpec((NB, d), lambda i: (i, 0)),
            pl.BlockSpec((d, 2 * hdk + hdv), lambda i: (0, 0)),
            pl.BlockSpec((1, 2 * hdk + hdv), lambda i: (0, 0)),
        ],
        out_specs=pl.BlockSpec((NB, 2 * hdk + hdv), lambda i: (i, 0)),
        out_shape=jax.ShapeDtypeStruct((n, 2 * hdk + hdv), jnp.float32),
    )(x, wqkv_t, bqkv)
    q = qkv[:, :hdk]
    k = qkv[:, hdk : 2 * hdk]
    v = qkv[:, 2 * hdk :]

    recv3 = edge_index[0].reshape(n_eb, 1, EB)
    send3 = edge_index[1].reshape(n_eb, 1, EB)

    idx_spec = pl.BlockSpec((1, 1, EB), lambda i: (i, 0, 0), memory_space=pltpu.SMEM)
    att, bmax = pl.pallas_call(
        _att_body,
        grid=(n_eb,),
        in_specs=[
            idx_spec,
            idx_spec,
            pl.BlockSpec((n, hdk), lambda i: (0, 0)),
            pl.BlockSpec((n, hdk), lambda i: (0, 0)),
        ],
        out_specs=[
            pl.BlockSpec((EB, H), lambda i: (i, 0)),
            pl.BlockSpec((1, 1, H), lambda i: (i, 0, 0)),
        ],
        out_shape=[
            jax.ShapeDtypeStruct((e, H), jnp.float32),
            jax.ShapeDtypeStruct((n_eb, 1, H), jnp.float32),
        ],
        scratch_shapes=[
            pltpu.VMEM((EB, hdk), jnp.float32),
            pltpu.VMEM((EB, hdk), jnp.float32),
        ],
    )(recv3, send3, q, k)

    attn, sums, rsums = pl.pallas_call(
        functools.partial(_sums_body, inv_sqrt_dk=1.0 / float(hdk // H) ** 0.5),
        grid=(n_eb,),
        in_specs=[
            idx_spec,
            pl.BlockSpec((EB, H), lambda i: (i, 0)),
            pl.BlockSpec((n_eb, 1, H), lambda i: (0, 0, 0)),
        ],
        out_specs=[
            pl.BlockSpec((EB, H), lambda i: (i, 0)),
            pl.BlockSpec((n, H), lambda i: (0, 0)),
            pl.BlockSpec((n, H), lambda i: (0, 0)),
        ],
        out_shape=[
            jax.ShapeDtypeStruct((e, H), jnp.float32),
            jax.ShapeDtypeStruct((n, H), jnp.float32),
            jax.ShapeDtypeStruct((n, H), jnp.float32),
        ],
    )(recv3, att, bmax)

    dv = hdv // H
    grp = NC * NS * SC_C
    ep = ((e + grp - 1) // grp) * grp
    per_w = ep // (NC * NS)
    redge = jnp.pad(edge_index, ((0, 0), (0, ep - e)))
    attn_p = jnp.pad(attn, ((0, ep - e), (0, 0)))
    vt2 = v.reshape(n, H, dv).transpose(1, 0, 2).reshape(H * n, dv)

    sc_msg = pl.kernel(
        functools.partial(_sc_msg_body, n=n, dv=dv, per_w=per_w),
        out_type=jax.ShapeDtypeStruct((NC, H, n, dv), jnp.float32),
        mesh=plsc.VectorSubcoreMesh(core_axis_name="c", subcore_axis_name="s"),
        compiler_params=pltpu.CompilerParams(needs_layout_passes=False),
        scratch_types=[
            pltpu.VMEM((n,), jnp.float32),
            pltpu.VMEM((2, SC_C), jnp.int32),
            pltpu.VMEM((2, SC_C), jnp.int32),
            pltpu.VMEM((SC_C,), jnp.int32),
            pltpu.VMEM((SC_C,), jnp.int32),
            pltpu.VMEM((SC_C,), jnp.float32),
            pltpu.VMEM((SC_C,), jnp.float32),
            pltpu.VMEM((16,), jnp.float32),
            pltpu.VMEM((SC_C, dv), jnp.float32),
            pltpu.VMEM((SC_C, dv), jnp.float32),
            pltpu.VMEM_SHARED((n, dv), jnp.float32),
            pltpu.SemaphoreType.DMA,
        ],
    )
    partials = sc_msg(vt2, redge, attn_p.T, rsums.T)
    outacc = partials.transpose(0, 2, 1, 3).reshape(NC, n, hdv)

    y = pl.pallas_call(
        _dense_body,
        grid=(n_nb,),
        in_specs=[
            pl.BlockSpec((NB, hdv), lambda i: (i, 0)),
            pl.BlockSpec((NB, hdv), lambda i: (i, 0)),
            pl.BlockSpec((hdv, d), lambda i: (0, 0)),
            pl.BlockSpec((1, d), lambda i: (0, 0)),
            pl.BlockSpec((d, d), lambda i: (0, 0)),
            pl.BlockSpec((1, d), lambda i: (0, 0)),
            pl.BlockSpec((NB, d), lambda i: (i, 0)),
        ],
        out_specs=pl.BlockSpec((NB, d), lambda i: (i, 0)),
        out_shape=jax.ShapeDtypeStruct((n, d), jnp.float32),
    )(outacc[0], outacc[1], Wagg.T, bagg[None, :], Wff.T, bff[None, :], x)
    return y


# speedup vs baseline: 5.5493x; 1.0370x over previous
import functools

import jax
import jax.numpy as jnp
from jax import lax
from jax.experimental import pallas as pl
from jax.experimental.pallas import tpu as pltpu
from jax.experimental.pallas import tpu_sc as plsc

H = 8
NB = 1000
EB = 2000
NC = 2
NS = 16
SC_C = 128


def _proj_body(x_ref, w_ref, b_ref, o_ref):
    o_ref[...] = (
        jnp.dot(x_ref[...], w_ref[...], preferred_element_type=jnp.float32)
        + b_ref[...]
    )


def _att_body(recv_ref, send_ref, q_ref, k_ref, att_ref, bmax_ref, qg_ref, kg_ref):
    dk = q_ref.shape[1] // H

    def gather(i, _):
        r = recv_ref[0, 0, i]
        s = send_ref[0, 0, i]
        qg_ref[pl.ds(i, 1), :] = q_ref[pl.ds(r, 1), :]
        kg_ref[pl.ds(i, 1), :] = k_ref[pl.ds(s, 1), :]
        return 0

    jax.lax.fori_loop(0, recv_ref.shape[2], gather, 0, unroll=16)
    p = qg_ref[...] * kg_ref[...]
    d_iota = jax.lax.broadcasted_iota(jnp.int32, (H * dk, H), 0)
    h_iota = jax.lax.broadcasted_iota(jnp.int32, (H * dk, H), 1)
    hsum = (d_iota // dk == h_iota).astype(jnp.float32)
    att = jnp.dot(p, hsum, preferred_element_type=jnp.float32)
    att_ref[...] = att
    bmax_ref[...] = jnp.max(att, axis=0, keepdims=True)[None]


def _sums_body(recv_ref, att_ref, bmax_ref, attn_ref, sums_ref, rsums_ref, *,
               inv_sqrt_dk):
    eb = pl.program_id(0)

    @pl.when(eb == 0)
    def _():
        sums_ref[...] = jnp.zeros_like(sums_ref)

    scale = 3.0 / jnp.max(bmax_ref[...])
    attn_ref[...] = jnp.exp(att_ref[...] * scale)

    def scatter(i, _):
        r = recv_ref[0, 0, i]
        sums_ref[pl.ds(r, 1), :] += attn_ref[pl.ds(i, 1), :]
        return 0

    jax.lax.fori_loop(0, recv_ref.shape[2], scatter, 0, unroll=16)

    @pl.when(eb == pl.num_programs(0) - 1)
    def _():
        rsums_ref[...] = inv_sqrt_dk / sums_ref[...]


def _sc_msg_body(vt_ref, ridx_ref, attn_ref, rs_hbm, out_ref,
                 rs_v, ridx0, ridx1, idx0, idx1, aw0, aw1, wbuf,
                 rows0, rows1, outsh, sem, *, n, dv, per_w):
    c = lax.axis_index("c")
    s = lax.axis_index("s")
    wbase = (c * NS + s) * per_w
    nr0 = (n // NS) & ~7
    nlast = n - nr0 * (NS - 1)
    assert SC_C % 16 == 0 and nr0 % 8 == 0 and nlast % 8 == 0
    base_row = s * nr0
    n_chunks = per_w // SC_C
    assert n_chunks % 2 == 0
    zeros16 = jnp.zeros((16,), jnp.float32)
    lane8 = dv // 16

    def head_body(h, _):
        pltpu.sync_copy(rs_hbm.at[h], rs_v)
        hn = h * n

        def zrows(i, _):
            for kk in range(lane8):
                rows0[i, pl.ds(kk * 16, 16)] = zeros16
            return 0

        jax.lax.fori_loop(0, SC_C, zrows, 0)

        def zero_range(total):
            off = 0
            while off < total:
                sz = min(SC_C, total - off)
                pltpu.sync_copy(
                    rows0.at[pl.ds(0, sz)],
                    outsh.at[pl.ds(base_row + off, sz)],
                )
                off += sz

        @pl.when(s < NS - 1)
        def _():
            zero_range(nr0)

        @pl.when(s == NS - 1)
        def _():
            zero_range(nlast)

        plsc.subcore_barrier()

        def load_edges(g, ridx_v, aw_v, idx_v):
            base = wbase + g * SC_C
            pltpu.sync_copy(ridx_ref.at[:, pl.ds(base, SC_C)], ridx_v)
            pltpu.sync_copy(attn_ref.at[h, pl.ds(base, SC_C)], aw_v)

            def mkidx(g2, _):
                sv = ridx_v[1, pl.ds(g2 * 16, 16)]
                idx_v[pl.ds(g2 * 16, 16)] = sv + hn
                return 0

            jax.lax.fori_loop(0, SC_C // 16, mkidx, 0)

        def process(ridx_v, aw_v, rows_v):
            def scale_grp(g2, _):
                rcv16 = ridx_v[0, pl.ds(g2 * 16, 16)]
                rs16 = plsc.load_gather(rs_v, [rcv16])
                aw16 = aw_v[pl.ds(g2 * 16, 16)]
                wbuf[...] = aw16 * rs16

                def scale_e(j, _):
                    wj = plsc.load_gather(wbuf, [jnp.broadcast_to(j, (16,))])
                    e = g2 * 16 + j
                    for kk in range(lane8):
                        rows_v[e, pl.ds(kk * 16, 16)] *= wj
                    return 0

                jax.lax.fori_loop(0, 16, scale_e, 0)
                return 0

            jax.lax.fori_loop(0, SC_C // 16, scale_grp, 0)
            pltpu.sync_copy(rows_v, outsh.at[ridx_v.at[0]], add=True)

        load_edges(0, ridx0, aw0, idx0)
        pltpu.async_copy(vt_ref.at[idx0], rows0, sem)

        def pair_body(t, _):
            pltpu.make_async_copy(vt_ref.at[idx0], rows0, sem).wait()
            load_edges(2 * t + 1, ridx1, aw1, idx1)
            pltpu.async_copy(vt_ref.at[idx1], rows1, sem)
            process(ridx0, aw0, rows0)
            pltpu.make_async_copy(vt_ref.at[idx1], rows1, sem).wait()

            @pl.when(t < n_chunks // 2 - 1)
            def _():
                load_edges(2 * t + 2, ridx0, aw0, idx0)
                pltpu.async_copy(vt_ref.at[idx0], rows0, sem)

            process(ridx1, aw1, rows1)
            return 0

        jax.lax.fori_loop(0, n_chunks // 2, pair_body, 0)
        plsc.subcore_barrier()

        @pl.when(s < NS - 1)
        def _():
            pltpu.sync_copy(
                outsh.at[pl.ds(base_row, nr0)],
                out_ref.at[c, h, pl.ds(base_row, nr0)],
            )

        @pl.when(s == NS - 1)
        def _():
            pltpu.sync_copy(
                outsh.at[pl.ds(base_row, nlast)],
                out_ref.at[c, h, pl.ds(base_row, nlast)],
            )

        plsc.subcore_barrier()
        return 0

    jax.lax.fori_loop(0, H, head_body, 0)


def _dense_body(p0_ref, p1_ref, waggt_ref, bagg_ref, wfft_ref, bff_ref, x_ref, y_ref):
    a = jax.nn.relu(p0_ref[...] + p1_ref[...])
    b = jax.nn.relu(
        jnp.dot(a, waggt_ref[...], preferred_element_type=jnp.float32) + bagg_ref[...]
    )
    c = jax.nn.relu(
        jnp.dot(b, wfft_ref[...], preferred_element_type=jnp.float32) + bff_ref[...]
    )
    y_ref[...] = x_ref[...] + c


def kernel(x, edge_index, Wk, bk, Wq, bq, Wv, bv, Wagg, bagg, Wff, bff):
    n, d = x.shape
    e = edge_index.shape[1]
    hdk = Wk.shape[0]
    hdv = Wv.shape[0]
    n_nb = n // NB
    n_eb = e // EB

    wqkv_t = jnp.concatenate([Wq, Wk, Wv], axis=0).T
    bqkv = jnp.concatenate([bq, bk, bv])[None, :]

    qkv = pl.pallas_call(
        _proj_body,
        grid=(n_nb,),
        in_specs=[
            pl.BlockSpec((NB, d), lambda i: (i, 0)),
            pl.BlockSpec((d, 2 * hdk + hdv), lambda i: (0, 0)),
            pl.BlockSpec((1, 2 * hdk + hdv), lambda i: (0, 0)),
        ],
        out_specs=pl.BlockSpec((NB, 2 * hdk + hdv), lambda i: (i, 0)),
        out_shape=jax.ShapeDtypeStruct((n, 2 * hdk + hdv), jnp.float32),
    )(x, wqkv_t, bqkv)
    q = qkv[:, :hdk]
    k = qkv[:, hdk : 2 * hdk]
    v = qkv[:, 2 * hdk :]

    recv3 = edge_index[0].reshape(n_eb, 1, EB)
    send3 = edge_index[1].reshape(n_eb, 1, EB)

    idx_spec = pl.BlockSpec((1, 1, EB), lambda i: (i, 0, 0), memory_space=pltpu.SMEM)
    att, bmax = pl.pallas_call(
        _att_body,
        grid=(n_eb,),
        in_specs=[
            idx_spec,
            idx_spec,
            pl.BlockSpec((n, hdk), lambda i: (0, 0)),
            pl.BlockSpec((n, hdk), lambda i: (0, 0)),
        ],
        out_specs=[
            pl.BlockSpec((EB, H), lambda i: (i, 0)),
            pl.BlockSpec((1, 1, H), lambda i: (i, 0, 0)),
        ],
        out_shape=[
            jax.ShapeDtypeStruct((e, H), jnp.float32),
            jax.ShapeDtypeStruct((n_eb, 1, H), jnp.float32),
        ],
        scratch_shapes=[
            pltpu.VMEM((EB, hdk), jnp.float32),
            pltpu.VMEM((EB, hdk), jnp.float32),
        ],
    )(recv3, send3, q, k)

    attn, sums, rsums = pl.pallas_call(
        functools.partial(_sums_body, inv_sqrt_dk=1.0 / float(hdk // H) ** 0.5),
        grid=(n_eb,),
        in_specs=[
            idx_spec,
            pl.BlockSpec((EB, H), lambda i: (i, 0)),
            pl.BlockSpec((n_eb, 1, H), lambda i: (0, 0, 0)),
        ],
        out_specs=[
            pl.BlockSpec((EB, H), lambda i: (i, 0)),
            pl.BlockSpec((n, H), lambda i: (0, 0)),
            pl.BlockSpec((n, H), lambda i: (0, 0)),
        ],
        out_shape=[
            jax.ShapeDtypeStruct((e, H), jnp.float32),
            jax.ShapeDtypeStruct((n, H), jnp.float32),
            jax.ShapeDtypeStruct((n, H), jnp.float32),
        ],
    )(recv3, att, bmax)

    dv = hdv // H
    grp = NC * NS * SC_C
    ep = ((e + grp - 1) // grp) * grp
    per_w = ep // (NC * NS)
    redge = jnp.pad(edge_index, ((0, 0), (0, ep - e)))
    attn_p = jnp.pad(attn, ((0, ep - e), (0, 0)))
    vt2 = v.reshape(n, H, dv).transpose(1, 0, 2).reshape(H * n, dv)

    sc_msg = pl.kernel(
        functools.partial(_sc_msg_body, n=n, dv=dv, per_w=per_w),
        out_type=jax.ShapeDtypeStruct((NC, H, n, dv), jnp.float32),
        mesh=plsc.VectorSubcoreMesh(core_axis_name="c", subcore_axis_name="s"),
        compiler_params=pltpu.CompilerParams(needs_layout_passes=False),
        scratch_types=[
            pltpu.VMEM((n,), jnp.float32),
            pltpu.VMEM((2, SC_C), jnp.int32),
            pltpu.VMEM((2, SC_C), jnp.int32),
            pltpu.VMEM((SC_C,), jnp.int32),
            pltpu.VMEM((SC_C,), jnp.int32),
            pltpu.VMEM((SC_C,), jnp.float32),
            pltpu.VMEM((SC_C,), jnp.float32),
            pltpu.VMEM((16,), jnp.float32),
            pltpu.VMEM((SC_C, dv), jnp.float32),
            pltpu.VMEM((SC_C, dv), jnp.float32),
            pltpu.VMEM_SHARED((n, dv), jnp.float32),
            pltpu.SemaphoreType.DMA,
        ],
    )
    partials = sc_msg(vt2, redge, attn_p.T, rsums.T)
    outacc = partials.transpose(0, 2, 1, 3).reshape(NC, n, hdv)

    y = pl.pallas_call(
        _dense_body,
        grid=(n_nb,),
        in_specs=[
            pl.BlockSpec((NB, hdv), lambda i: (i, 0)),
            pl.BlockSpec((NB, hdv), lambda i: (i, 0)),
            pl.BlockSpec((hdv, d), lambda i: (0, 0)),
            pl.BlockSpec((1, d), lambda i: (0, 0)),
            pl.BlockSpec((d, d), lambda i: (0, 0)),
            pl.BlockSpec((1, d), lambda i: (0, 0)),
            pl.BlockSpec((NB, d), lambda i: (i, 0)),
        ],
        out_specs=pl.BlockSpec((NB, d), lambda i: (i, 0)),
        out_shape=jax.ShapeDtypeStruct((n, d), jnp.float32),
    )(outacc[0], outacc[1], Wagg.T, bagg[None, :], Wff.T, bff[None, :], x)
    return y


# speedup vs baseline: 5.5555x; 1.0011x over previous
import functools

import jax
import jax.numpy as jnp
from jax import lax
from jax.experimental import pallas as pl
from jax.experimental.pallas import tpu as pltpu
from jax.experimental.pallas import tpu_sc as plsc

H = 8
NB = 1000
EB = 2000
NC = 2
NS = 16
SC_C = 128


def _proj_body(x_ref, w_ref, b_ref, o_ref):
    o_ref[...] = (
        jnp.dot(x_ref[...], w_ref[...], preferred_element_type=jnp.float32)
        + b_ref[...]
    )


def _att_body(recv_ref, send_ref, q_ref, k_ref, att_ref, bmax_ref, qg_ref, kg_ref):
    dk = q_ref.shape[1] // H

    def gather(i, _):
        r = recv_ref[0, 0, i]
        s = send_ref[0, 0, i]
        qg_ref[pl.ds(i, 1), :] = q_ref[pl.ds(r, 1), :] * k_ref[pl.ds(s, 1), :]
        return 0

    jax.lax.fori_loop(0, recv_ref.shape[2], gather, 0, unroll=16)
    p = qg_ref[...]
    d_iota = jax.lax.broadcasted_iota(jnp.int32, (H * dk, H), 0)
    h_iota = jax.lax.broadcasted_iota(jnp.int32, (H * dk, H), 1)
    hsum = (d_iota // dk == h_iota).astype(jnp.float32)
    att = jnp.dot(p, hsum, preferred_element_type=jnp.float32)
    att_ref[...] = att
    bmax_ref[...] = jnp.max(att, axis=0, keepdims=True)[None]


def _sums_body(recv_ref, att_ref, bmax_ref, attn_ref, sums_ref, rsums_ref, *,
               inv_sqrt_dk):
    eb = pl.program_id(0)

    @pl.when(eb == 0)
    def _():
        sums_ref[...] = jnp.zeros_like(sums_ref)

    scale = 3.0 / jnp.max(bmax_ref[...])
    attn_ref[...] = jnp.exp(att_ref[...] * scale)

    def scatter(i, _):
        r = recv_ref[0, 0, i]
        sums_ref[pl.ds(r, 1), :] += attn_ref[pl.ds(i, 1), :]
        return 0

    jax.lax.fori_loop(0, recv_ref.shape[2], scatter, 0, unroll=16)

    @pl.when(eb == pl.num_programs(0) - 1)
    def _():
        rsums_ref[...] = inv_sqrt_dk / sums_ref[...]


def _sc_msg_body(vt_ref, ridx_ref, attn_ref, rs_hbm, out_ref,
                 rs_v, ridx0, ridx1, idx0, idx1, aw0, aw1, wbuf,
                 rows0, rows1, outsh, sem, *, n, dv, per_w):
    c = lax.axis_index("c")
    s = lax.axis_index("s")
    wbase = (c * NS + s) * per_w
    nr0 = (n // NS) & ~7
    nlast = n - nr0 * (NS - 1)
    assert SC_C % 16 == 0 and nr0 % 8 == 0 and nlast % 8 == 0
    base_row = s * nr0
    n_chunks = per_w // SC_C
    assert n_chunks % 2 == 0
    zeros16 = jnp.zeros((16,), jnp.float32)
    lane8 = dv // 16

    def head_body(h, _):
        pltpu.sync_copy(rs_hbm.at[h], rs_v)
        hn = h * n

        def zrows(i, _):
            for kk in range(lane8):
                rows0[i, pl.ds(kk * 16, 16)] = zeros16
            return 0

        jax.lax.fori_loop(0, SC_C, zrows, 0)

        def zero_range(total):
            off = 0
            while off < total:
                sz = min(SC_C, total - off)
                pltpu.sync_copy(
                    rows0.at[pl.ds(0, sz)],
                    outsh.at[pl.ds(base_row + off, sz)],
                )
                off += sz

        @pl.when(s < NS - 1)
        def _():
            zero_range(nr0)

        @pl.when(s == NS - 1)
        def _():
            zero_range(nlast)

        plsc.subcore_barrier()

        def load_edges(g, ridx_v, aw_v, idx_v):
            base = wbase + g * SC_C
            pltpu.sync_copy(ridx_ref.at[:, pl.ds(base, SC_C)], ridx_v)
            pltpu.sync_copy(attn_ref.at[h, pl.ds(base, SC_C)], aw_v)

            def mkidx(g2, _):
                sv = ridx_v[1, pl.ds(g2 * 16, 16)]
                idx_v[pl.ds(g2 * 16, 16)] = sv + hn
                return 0

            jax.lax.fori_loop(0, SC_C // 16, mkidx, 0)

        def process(ridx_v, aw_v, rows_v):
            def scale_grp(g2, _):
                rcv16 = ridx_v[0, pl.ds(g2 * 16, 16)]
                rs16 = plsc.load_gather(rs_v, [rcv16])
                aw16 = aw_v[pl.ds(g2 * 16, 16)]
                wbuf[...] = aw16 * rs16

                def scale_e(j, _):
                    wj = plsc.load_gather(wbuf, [jnp.broadcast_to(j, (16,))])
                    e = g2 * 16 + j
                    for kk in range(lane8):
                        rows_v[e, pl.ds(kk * 16, 16)] *= wj
                    return 0

                jax.lax.fori_loop(0, 16, scale_e, 0)
                return 0

            jax.lax.fori_loop(0, SC_C // 16, scale_grp, 0)
            pltpu.sync_copy(rows_v, outsh.at[ridx_v.at[0]], add=True)

        load_edges(0, ridx0, aw0, idx0)
        pltpu.async_copy(vt_ref.at[idx0], rows0, sem)

        def pair_body(t, _):
            pltpu.make_async_copy(vt_ref.at[idx0], rows0, sem).wait()
            load_edges(2 * t + 1, ridx1, aw1, idx1)
            pltpu.async_copy(vt_ref.at[idx1], rows1, sem)
            process(ridx0, aw0, rows0)
            pltpu.make_async_copy(vt_ref.at[idx1], rows1, sem).wait()

            @pl.when(t < n_chunks // 2 - 1)
            def _():
                load_edges(2 * t + 2, ridx0, aw0, idx0)
                pltpu.async_copy(vt_ref.at[idx0], rows0, sem)

            process(ridx1, aw1, rows1)
            return 0

        jax.lax.fori_loop(0, n_chunks // 2, pair_body, 0)
        plsc.subcore_barrier()

        @pl.when(s < NS - 1)
        def _():
            pltpu.sync_copy(
                outsh.at[pl.ds(base_row, nr0)],
                out_ref.at[c, h, pl.ds(base_row, nr0)],
            )

        @pl.when(s == NS - 1)
        def _():
            pltpu.sync_copy(
                outsh.at[pl.ds(base_row, nlast)],
                out_ref.at[c, h, pl.ds(base_row, nlast)],
            )

        plsc.subcore_barrier()
        return 0

    jax.lax.fori_loop(0, H, head_body, 0)


def _dense_body(p0_ref, p1_ref, waggt_ref, bagg_ref, wfft_ref, bff_ref, x_ref, y_ref):
    a = jax.nn.relu(p0_ref[...] + p1_ref[...])
    b = jax.nn.relu(
        jnp.dot(a, waggt_ref[...], preferred_element_type=jnp.float32) + bagg_ref[...]
    )
    c = jax.nn.relu(
        jnp.dot(b, wfft_ref[...], preferred_element_type=jnp.float32) + bff_ref[...]
    )
    y_ref[...] = x_ref[...] + c


def kernel(x, edge_index, Wk, bk, Wq, bq, Wv, bv, Wagg, bagg, Wff, bff):
    n, d = x.shape
    e = edge_index.shape[1]
    hdk = Wk.shape[0]
    hdv = Wv.shape[0]
    n_nb = n // NB
    n_eb = e // EB

    wqkv_t = jnp.concatenate([Wq, Wk, Wv], axis=0).T
    bqkv = jnp.concatenate([bq, bk, bv])[None, :]

    qkv = pl.pallas_call(
        _proj_body,
        grid=(n_nb,),
        in_specs=[
            pl.BlockSpec((NB, d), lambda i: (i, 0)),
            pl.BlockSpec((d, 2 * hdk + hdv), lambda i: (0, 0)),
            pl.BlockSpec((1, 2 * hdk + hdv), lambda i: (0, 0)),
        ],
        out_specs=pl.BlockSpec((NB, 2 * hdk + hdv), lambda i: (i, 0)),
        out_shape=jax.ShapeDtypeStruct((n, 2 * hdk + hdv), jnp.float32),
    )(x, wqkv_t, bqkv)
    q = qkv[:, :hdk]
    k = qkv[:, hdk : 2 * hdk]
    v = qkv[:, 2 * hdk :]

    recv3 = edge_index[0].reshape(n_eb, 1, EB)
    send3 = edge_index[1].reshape(n_eb, 1, EB)

    idx_spec = pl.BlockSpec((1, 1, EB), lambda i: (i, 0, 0), memory_space=pltpu.SMEM)
    att, bmax = pl.pallas_call(
        _att_body,
        grid=(n_eb,),
        in_specs=[
            idx_spec,
            idx_spec,
            pl.BlockSpec((n, hdk), lambda i: (0, 0)),
            pl.BlockSpec((n, hdk), lambda i: (0, 0)),
        ],
        out_specs=[
            pl.BlockSpec((EB, H), lambda i: (i, 0)),
            pl.BlockSpec((1, 1, H), lambda i: (i, 0, 0)),
        ],
        out_shape=[
            jax.ShapeDtypeStruct((e, H), jnp.float32),
            jax.ShapeDtypeStruct((n_eb, 1, H), jnp.float32),
        ],
        scratch_shapes=[
            pltpu.VMEM((EB, hdk), jnp.float32),
            pltpu.VMEM((EB, hdk), jnp.float32),
        ],
    )(recv3, send3, q, k)

    attn, sums, rsums = pl.pallas_call(
        functools.partial(_sums_body, inv_sqrt_dk=1.0 / float(hdk // H) ** 0.5),
        grid=(n_eb,),
        in_specs=[
            idx_spec,
            pl.BlockSpec((EB, H), lambda i: (i, 0)),
            pl.BlockSpec((n_eb, 1, H), lambda i: (0, 0, 0)),
        ],
        out_specs=[
            pl.BlockSpec((EB, H), lambda i: (i, 0)),
            pl.BlockSpec((n, H), lambda i: (0, 0)),
            pl.BlockSpec((n, H), lambda i: (0, 0)),
        ],
        out_shape=[
            jax.ShapeDtypeStruct((e, H), jnp.float32),
            jax.ShapeDtypeStruct((n, H), jnp.float32),
            jax.ShapeDtypeStruct((n, H), jnp.float32),
        ],
    )(recv3, att, bmax)

    dv = hdv // H
    grp = NC * NS * SC_C
    ep = ((e + grp - 1) // grp) * grp
    per_w = ep // (NC * NS)
    redge = jnp.pad(edge_index, ((0, 0), (0, ep - e)))
    attn_p = jnp.pad(attn, ((0, ep - e), (0, 0)))
    vt2 = v.reshape(n, H, dv).transpose(1, 0, 2).reshape(H * n, dv)

    sc_msg = pl.kernel(
        functools.partial(_sc_msg_body, n=n, dv=dv, per_w=per_w),
        out_type=jax.ShapeDtypeStruct((NC, H, n, dv), jnp.float32),
        mesh=plsc.VectorSubcoreMesh(core_axis_name="c", subcore_axis_name="s"),
        compiler_params=pltpu.CompilerParams(needs_layout_passes=False),
        scratch_types=[
            pltpu.VMEM((n,), jnp.float32),
            pltpu.VMEM((2, SC_C), jnp.int32),
            pltpu.VMEM((2, SC_C), jnp.int32),
            pltpu.VMEM((SC_C,), jnp.int32),
            pltpu.VMEM((SC_C,), jnp.int32),
            pltpu.VMEM((SC_C,), jnp.float32),
            pltpu.VMEM((SC_C,), jnp.float32),
            pltpu.VMEM((16,), jnp.float32),
            pltpu.VMEM((SC_C, dv), jnp.float32),
            pltpu.VMEM((SC_C, dv), jnp.float32),
            pltpu.VMEM_SHARED((n, dv), jnp.float32),
            pltpu.SemaphoreType.DMA,
        ],
    )
    partials = sc_msg(vt2, redge, attn_p.T, rsums.T)
    outacc = partials.transpose(0, 2, 1, 3).reshape(NC, n, hdv)

    y = pl.pallas_call(
        _dense_body,
        grid=(n_nb,),
        in_specs=[
            pl.BlockSpec((NB, hdv), lambda i: (i, 0)),
            pl.BlockSpec((NB, hdv), lambda i: (i, 0)),
            pl.BlockSpec((hdv, d), lambda i: (0, 0)),
            pl.BlockSpec((1, d), lambda i: (0, 0)),
            pl.BlockSpec((d, d), lambda i: (0, 0)),
            pl.BlockSpec((1, d), lambda i: (0, 0)),
            pl.BlockSpec((NB, d), lambda i: (i, 0)),
        ],
        out_specs=pl.BlockSpec((NB, d), lambda i: (i, 0)),
        out_shape=jax.ShapeDtypeStruct((n, d), jnp.float32),
    )(outacc[0], outacc[1], Wagg.T, bagg[None, :], Wff.T, bff[None, :], x)
    return y
